# Initial kernel scaffold; baseline (speedup 1.0000x reference)
import jax
import jax.numpy as jnp
from jax.experimental import pallas as pl


def kernel(x, fc_weight, bias, offsets):
    raise NotImplementedError("write your pallas kernel here")


# speedup vs baseline: 1.1137x; 1.1137x over previous
import jax
import jax.numpy as jnp
from jax import lax
from jax.experimental import pallas as pl
from jax.experimental.pallas import tpu as pltpu
from jax.experimental.pallas import tpu_sc as plsc

BATCH = 16384
NF = 26
LANES = 16
NC = 2
NS = 16
NW = NC * NS
ROWS_PER_W = BATCH // NW
ELEMS_PER_W = ROWS_PER_W * NF
CHUNKS = ELEMS_PER_W // LANES
GCHUNK = 128
NGATHER = ELEMS_PER_W // GCHUNK
OFFS_PAD = 32


def _lr_body(x_hbm, table_hbm, offs_hbm, bias_hbm, out_hbm,
             x_v, emb_v, offs_v, bias_v, out_v, sem):
    wid = lax.axis_index("s") * NC + lax.axis_index("c")
    base = wid * ELEMS_PER_W

    pltpu.sync_copy(x_hbm.at[pl.ds(base, ELEMS_PER_W)], x_v)
    pltpu.sync_copy(offs_hbm, offs_v)
    pltpu.sync_copy(bias_hbm, bias_v)

    lane = lax.iota(jnp.int32, LANES)

    def idx_body(c, f_vec):
        off = plsc.load_gather(offs_v, [f_vec])
        sl = pl.ds(c * LANES, LANES)
        x_v[sl] = x_v[sl] + off
        f2 = f_vec + LANES
        return jnp.where(f2 >= NF, f2 - NF, f2)

    lax.fori_loop(0, CHUNKS, idx_body, lane)

    def fire(j, carry):
        sl = pl.ds(j * GCHUNK, GCHUNK)
        pltpu.async_copy(table_hbm.at[x_v.at[sl]], emb_v.at[sl], sem)
        return carry

    lax.fori_loop(0, NGATHER, fire, 0)
    pltpu.make_async_copy(table_hbm.at[x_v], emb_v, sem).wait()

    bias_vec = bias_v[...]

    def red_body(c, carry):
        gbase = (c * LANES + lane) * NF

        def f_body(f, acc):
            return acc + plsc.load_gather(emb_v, [gbase + f])

        s = lax.fori_loop(0, NF, f_body, jnp.zeros((LANES,), jnp.float32))
        s = s + bias_vec
        out_v[pl.ds(c * LANES, LANES)] = 1.0 / (1.0 + jnp.exp(-s))
        return carry

    lax.fori_loop(0, ROWS_PER_W // LANES, red_body, 0)

    pltpu.sync_copy(out_v, out_hbm.at[pl.ds(wid * ROWS_PER_W, ROWS_PER_W)])


def kernel(x, fc_weight, bias, offsets):
    x_flat = x.reshape(-1)
    table = fc_weight.reshape(-1)
    offs_pad = jnp.pad(offsets, (0, OFFS_PAD - NF))
    bias_pad = jnp.broadcast_to(bias, (LANES,)).astype(jnp.float32)

    mesh = plsc.VectorSubcoreMesh(core_axis_name="c", subcore_axis_name="s")
    run = pl.kernel(
        _lr_body,
        out_type=jax.ShapeDtypeStruct((BATCH,), jnp.float32),
        mesh=mesh,
        compiler_params=pltpu.CompilerParams(needs_layout_passes=False),
        scratch_types=[
            pltpu.VMEM((ELEMS_PER_W,), jnp.int32),
            pltpu.VMEM((ELEMS_PER_W,), jnp.float32),
            pltpu.VMEM((OFFS_PAD,), jnp.int32),
            pltpu.VMEM((LANES,), jnp.float32),
            pltpu.VMEM((ROWS_PER_W,), jnp.float32),
            pltpu.SemaphoreType.DMA,
        ],
    )
    return run(x_flat, table, offs_pad, bias_pad)


# speedup vs baseline: 1.2260x; 1.1009x over previous
import jax
import jax.numpy as jnp
from jax import lax
from jax.experimental import pallas as pl
from jax.experimental.pallas import tpu as pltpu
from jax.experimental.pallas import tpu_sc as plsc

BATCH = 16384
NF = 26
LANES = 16
NC = 2
NS = 16
NW = NC * NS
ROWS_PER_W = BATCH // NW
ELEMS_PER_W = ROWS_PER_W * NF
BCHUNKS = ROWS_PER_W // LANES
GCHUNK = 128
NGATHER = ELEMS_PER_W // GCHUNK
OFFS_PAD = 32


def _lr_body(xt_hbm, table_hbm, offs_hbm, bias_hbm, out_hbm,
             x_v, idx_v, emb_v, offs_v, bias_v, out_v, sem):
    wid = lax.axis_index("s") * NC + lax.axis_index("c")
    row0 = wid * ROWS_PER_W

    pltpu.sync_copy(xt_hbm.at[:, pl.ds(row0, ROWS_PER_W)], x_v)
    pltpu.sync_copy(offs_hbm, offs_v)
    pltpu.sync_copy(bias_hbm, bias_v)

    def idx_f(f, carry):
        off = plsc.load_gather(offs_v, [jnp.full((LANES,), f, jnp.int32)])

        def idx_b(c, carry2):
            idx_v[pl.ds(f * ROWS_PER_W + c * LANES, LANES)] = (
                x_v[f, pl.ds(c * LANES, LANES)] + off)
            return carry2

        lax.fori_loop(0, BCHUNKS, idx_b, 0)
        return carry

    lax.fori_loop(0, NF, idx_f, 0)

    def fire(j, carry):
        sl = pl.ds(j * GCHUNK, GCHUNK)
        pltpu.async_copy(table_hbm.at[idx_v.at[sl]], emb_v.at[sl], sem)
        return carry

    lax.fori_loop(0, NGATHER, fire, 0)
    pltpu.make_async_copy(table_hbm.at[idx_v], emb_v, sem).wait()

    bias_vec = bias_v[...]

    def red_body(c, carry):
        def f_body(f, acc):
            return acc + emb_v[pl.ds(f * ROWS_PER_W + c * LANES, LANES)]

        s = lax.fori_loop(0, NF, f_body, jnp.zeros((LANES,), jnp.float32))
        s = s + bias_vec
        out_v[pl.ds(c * LANES, LANES)] = 1.0 / (1.0 + jnp.exp(-s))
        return carry

    lax.fori_loop(0, BCHUNKS, red_body, 0)

    pltpu.sync_copy(out_v, out_hbm.at[pl.ds(row0, ROWS_PER_W)])


def kernel(x, fc_weight, bias, offsets):
    xt = x.T
    table = fc_weight.reshape(-1)
    offs_pad = jnp.pad(offsets, (0, OFFS_PAD - NF))
    bias_pad = jnp.broadcast_to(bias, (LANES,)).astype(jnp.float32)

    mesh = plsc.VectorSubcoreMesh(core_axis_name="c", subcore_axis_name="s")
    run = pl.kernel(
        _lr_body,
        out_type=jax.ShapeDtypeStruct((BATCH,), jnp.float32),
        mesh=mesh,
        compiler_params=pltpu.CompilerParams(
            needs_layout_passes=False, use_tc_tiling_on_sc=False),
        scratch_types=[
            pltpu.VMEM((NF, ROWS_PER_W), jnp.int32),
            pltpu.VMEM((ELEMS_PER_W,), jnp.int32),
            pltpu.VMEM((ELEMS_PER_W,), jnp.float32),
            pltpu.VMEM((OFFS_PAD,), jnp.int32),
            pltpu.VMEM((LANES,), jnp.float32),
            pltpu.VMEM((ROWS_PER_W,), jnp.float32),
            pltpu.SemaphoreType.DMA,
        ],
    )
    return run(xt, table, offs_pad, bias_pad)


# speedup vs baseline: 1.2786x; 1.0429x over previous
import jax
import jax.numpy as jnp
from jax import lax
from jax.experimental import pallas as pl
from jax.experimental.pallas import tpu as pltpu
from jax.experimental.pallas import tpu_sc as plsc

BATCH = 16384
NF = 26
LANES = 16
NC = 2
NS = 16
NW = NC * NS
ROWS_PER_W = BATCH // NW
ELEMS_PER_W = ROWS_PER_W * NF
BCHUNKS = ROWS_PER_W // LANES
GCHUNK = 128
FIELD_DMAS = ROWS_PER_W // GCHUNK
OFFS_PAD = 32


def _lr_body(xt_hbm, table_hbm, offs_hbm, bias_hbm, out_hbm,
             x_v, idx_v, emb_v, offs_v, bias_v, out_v, sem):
    wid = lax.axis_index("s") * NC + lax.axis_index("c")
    row0 = wid * ROWS_PER_W

    pltpu.sync_copy(xt_hbm.at[:, pl.ds(row0, ROWS_PER_W)], x_v)
    pltpu.sync_copy(offs_hbm, offs_v.at[pl.ds(0, NF)])
    pltpu.sync_copy(bias_hbm, bias_v.at[pl.ds(0, 1)])

    zero16 = jnp.zeros((LANES,), jnp.int32)
    bias_vec = plsc.load_gather(bias_v, [zero16])

    def idx_f(f, carry):
        off = plsc.load_gather(offs_v, [jnp.full((LANES,), f, jnp.int32)])

        def idx_b(c, carry2):
            idx_v[pl.ds(f * ROWS_PER_W + c * LANES, LANES)] = (
                x_v[f, pl.ds(c * LANES, LANES)] + off)
            return carry2

        lax.fori_loop(0, BCHUNKS, idx_b, 0)

        def fire_j(j, carry3):
            sl = pl.ds(f * ROWS_PER_W + j * GCHUNK, GCHUNK)
            pltpu.async_copy(table_hbm.at[idx_v.at[sl]], emb_v.at[sl], sem)
            return carry3

        lax.fori_loop(0, FIELD_DMAS, fire_j, 0)
        return carry

    lax.fori_loop(0, NF, idx_f, 0)

    pltpu.make_async_copy(table_hbm.at[idx_v], emb_v, sem).wait()

    def red_body(c, carry):
        def f_body(f, acc):
            return acc + emb_v[pl.ds(f * ROWS_PER_W + c * LANES, LANES)]

        s = lax.fori_loop(0, NF, f_body, jnp.zeros((LANES,), jnp.float32))
        s = s + bias_vec
        out_v[pl.ds(c * LANES, LANES)] = 1.0 / (1.0 + jnp.exp(-s))
        return carry

    lax.fori_loop(0, BCHUNKS, red_body, 0)

    pltpu.sync_copy(out_v, out_hbm.at[pl.ds(row0, ROWS_PER_W)])


def kernel(x, fc_weight, bias, offsets):
    xt = x.T
    table = fc_weight.reshape(-1)

    mesh = plsc.VectorSubcoreMesh(core_axis_name="c", subcore_axis_name="s")
    run = pl.kernel(
        _lr_body,
        out_type=jax.ShapeDtypeStruct((BATCH,), jnp.float32),
        mesh=mesh,
        compiler_params=pltpu.CompilerParams(
            needs_layout_passes=False, use_tc_tiling_on_sc=False),
        scratch_types=[
            pltpu.VMEM((NF, ROWS_PER_W), jnp.int32),
            pltpu.VMEM((ELEMS_PER_W,), jnp.int32),
            pltpu.VMEM((ELEMS_PER_W,), jnp.float32),
            pltpu.VMEM((OFFS_PAD,), jnp.int32),
            pltpu.VMEM((LANES,), jnp.float32),
            pltpu.VMEM((ROWS_PER_W,), jnp.float32),
            pltpu.SemaphoreType.DMA,
        ],
    )
    return run(xt, table, offsets, bias)
